# baseline (speedup 1.0000x reference)
import functools

import jax
import jax.numpy as jnp
from jax import lax
from jax.experimental import pallas as pl
from jax.experimental.pallas import tpu as pltpu
from jax.experimental.pallas import tpu_sc as plsc

B, Lq, Ld, Eq, Ed = 64, 20, 500, 10, 100
DW, DE = 300, 100
KW, KE = 20, 10

_N_DE = B * Ld
_N_QE = B * Lq
_N_DEE = B * Ed
_N_QEE = B * Eq
_NWORK = 32


def _row_gather(idx_hbm, tab, out, idx_v, sem, base, n):
    pltpu.sync_copy(idx_hbm.at[pl.ds(base, n)], idx_v.at[pl.ds(0, n)])
    nfull = n // 16

    def fire_at(st):
        v = idx_v[pl.ds(st, 16)]
        for j in range(16):
            pltpu.make_async_copy(tab.at[pl.ds(v[j], 1)],
                                  out.at[pl.ds(base + st + j, 1)], sem).start()

    def drain16():
        for _ in range(16):
            pltpu.make_async_copy(tab.at[pl.ds(0, 1)],
                                  out.at[pl.ds(base, 1)], sem).wait()

    fire_at(0)

    def body(k, _):
        fire_at((k + 1) * 16)
        drain16()
        return 0
    lax.fori_loop(0, nfull - 1, body, 0)
    if n % 16:
        fire_at(n - 16)
        drain16()
    drain16()


def _sc_gather_fn(doc_tok, qrl_tok, docs_e, qrls_e, wtab, etab, itab,
                  de_out, qe_out, dee_out, qee_out, idf_out, idx_v, sem):
    c = lax.axis_index("c")
    s = lax.axis_index("s")
    wid = s * 2 + c

    _row_gather(doc_tok, wtab, de_out, idx_v, sem,
                wid * (_N_DE // _NWORK), _N_DE // _NWORK)
    _row_gather(qrl_tok, wtab, qe_out, idx_v, sem,
                wid * (_N_QE // _NWORK), _N_QE // _NWORK)
    _row_gather(qrl_tok, itab, idf_out, idx_v, sem,
                wid * (_N_QE // _NWORK), _N_QE // _NWORK)
    _row_gather(docs_e, etab, dee_out, idx_v, sem,
                wid * (_N_DEE // _NWORK), _N_DEE // _NWORK)

    @pl.when(wid < 8)
    def _():
        _row_gather(qrls_e, etab, qee_out, idx_v, sem, wid * 80, 80)


def _sc_gather(doc_tok, qrl_tok, docs_e, qrls_e, wtab, etab, itab):
    f32 = jnp.float32
    mesh = plsc.VectorSubcoreMesh(core_axis_name="c", subcore_axis_name="s")
    call = functools.partial(
        pl.kernel,
        mesh=mesh,
        out_type=(
            jax.ShapeDtypeStruct((_N_DE, DW), f32),
            jax.ShapeDtypeStruct((_N_QE, DW), f32),
            jax.ShapeDtypeStruct((_N_DEE, DE), f32),
            jax.ShapeDtypeStruct((_N_QEE, DE), f32),
            jax.ShapeDtypeStruct((_N_QE, 1), f32),
        ),
        scratch_types=[
            pltpu.VMEM((_N_DE // _NWORK,), jnp.int32),
            pltpu.SemaphoreType.DMA,
        ],
    )
    return call(_sc_gather_fn)(doc_tok, qrl_tok, docs_e, qrls_e, wtab, etab, itab)


def _topk_rows(mat, k):
    r, c = mat.shape
    col = lax.broadcasted_iota(jnp.int32, (r, c), 1)
    outs = []
    x = mat
    for _ in range(k):
        m = jnp.max(x, axis=1, keepdims=True)
        first = jnp.min(jnp.where(x == m, col, c), axis=1, keepdims=True)
        outs.append(m)
        x = jnp.where(col == first, -jnp.inf, x)
    return jnp.concatenate(outs, axis=1)


def _tc_body(ids_ref, qe_ref, de_ref, dee_ref, qee_ref, idf_ref,
             aw_ref, ae_ref, g1w_ref, g1b_ref, g3w_ref, g3b_ref,
             g2w_ref, g2b_ref, g4w_ref, g4b_ref,
             w1_ref, b1_ref, w2_ref, b2_ref, w3_ref, b3_ref,
             w4_ref, b4_ref, w5_ref, b5_ref, out_ref):
    f32 = jnp.float32

    def dot(a_, b_):
        return lax.dot_general(a_, b_, (((1,), (0,)), ((), ())),
                               preferred_element_type=f32)

    def dot_t(a_, b_):
        return lax.dot_general(a_, b_, (((1,), (1,)), ((), ())),
                               preferred_element_type=f32)

    x20 = qe_ref[0]
    d500 = de_ref[0]
    adj = aw_ref[0]

    f0 = dot_t(d500, x20)
    sim = f0.T

    def ggnn(x, wref, bref):
        a = dot(adj, x)
        w = wref[...]
        bb = bref[...]
        z = jax.nn.sigmoid(dot(a, w[0]) + bb[0] + dot(x, w[1]) + bb[1])
        rr = jax.nn.sigmoid(dot(a, w[2]) + bb[2] + dot(x, w[3]) + bb[3])
        h = jnp.maximum(dot(a, w[4]) + bb[4] + dot(rr * x, w[5]) + bb[5], 0.0)
        return h * z + x * (1.0 - z)

    f1 = ggnn(f0, g1w_ref, g1b_ref)
    f2 = ggnn(f1, g3w_ref, g3b_ref)

    stack = jnp.concatenate([sim, f1.T, f2.T], axis=0)
    ks = _topk_rows(stack, KW)
    wf = jnp.concatenate([ks[0:20], ks[20:40], ks[40:60]], axis=1)

    h = jnp.maximum(dot(wf, w1_ref[...]) + b1_ref[...], 0.0)
    h = jnp.maximum(dot(h, w2_ref[...]) + b2_ref[...], 0.0)
    ws = dot(h, w3_ref[...]) + b3_ref[...]
    word_score = jnp.sum(idf_ref[0] * ws)

    qet = qee_ref[0]
    det = dee_ref[0]
    adje = ae_ref[0]
    sime = dot_t(qet, det)
    g0 = jnp.max(sime, axis=0, keepdims=True)
    gw2 = g2w_ref[...]
    gb2 = g2b_ref[...]
    gw4 = g4w_ref[...]
    gb4 = g4b_ref[...]

    def ggnn_s(g, w, bb):
        a = dot_t(g, adje)
        z = jax.nn.sigmoid(a * w[:, 0:1] + bb[:, 0:1] + g * w[:, 1:2] + bb[:, 1:2])
        rr = jax.nn.sigmoid(a * w[:, 2:3] + bb[:, 2:3] + g * w[:, 3:4] + bb[:, 3:4])
        h_ = jnp.maximum(a * w[:, 4:5] + bb[:, 4:5] + (rr * g) * w[:, 5:6] + bb[:, 5:6], 0.0)
        return h_ * z + g * (1.0 - z)

    g1 = ggnn_s(g0, gw2, gb2)
    g2 = ggnn_s(g1, gw4, gb4)
    ge = jnp.concatenate([g0, g1, g2], axis=0)
    ek = _topk_rows(ge, KE)
    ef = jnp.concatenate([ek[0:1], ek[1:2], ek[2:3]], axis=1)
    eh = jnp.maximum(dot(ef, w4_ref[...]) + b4_ref[...], 0.0)
    es = dot(eh, w5_ref[...]) + b5_ref[...]

    out_ref[...] = (word_score + es)[None]


def _tc_call(doc_ids, qe3, de3, dee3, qee3, idf3, word_adj, ent_adj,
             g1w, g1b, g3w, g3b, g2w, g2b, g4w, g4b,
             w1, b1, w2, b2, w3, b3, w4, b4, w5, b5):
    fixed = lambda *_: tuple(0 for _ in range(3))
    fixed2 = lambda *_: (0, 0)
    grid_spec = pltpu.PrefetchScalarGridSpec(
        num_scalar_prefetch=1,
        grid=(B,),
        in_specs=[
            pl.BlockSpec((1, Lq, DW), lambda b, ids: (b, 0, 0)),
            pl.BlockSpec((1, Ld, DW), lambda b, ids: (b, 0, 0)),
            pl.BlockSpec((1, Ed, DE), lambda b, ids: (b, 0, 0)),
            pl.BlockSpec((1, Eq, DE), lambda b, ids: (b, 0, 0)),
            pl.BlockSpec((1, Lq, 1), lambda b, ids: (b, 0, 0)),
            pl.BlockSpec((1, Ld, Ld), lambda b, ids: (ids[b], 0, 0)),
            pl.BlockSpec((1, Ed, Ed), lambda b, ids: (ids[b], 0, 0)),
            pl.BlockSpec((6, Lq, Lq), fixed),
            pl.BlockSpec((6, 1, Lq), fixed),
            pl.BlockSpec((6, Lq, Lq), fixed),
            pl.BlockSpec((6, 1, Lq), fixed),
            pl.BlockSpec((1, 6), fixed2),
            pl.BlockSpec((1, 6), fixed2),
            pl.BlockSpec((1, 6), fixed2),
            pl.BlockSpec((1, 6), fixed2),
            pl.BlockSpec((3 * KW, 64), fixed2),
            pl.BlockSpec((1, 64), fixed2),
            pl.BlockSpec((64, 32), fixed2),
            pl.BlockSpec((1, 32), fixed2),
            pl.BlockSpec((32, 1), fixed2),
            pl.BlockSpec((1, 1), fixed2),
            pl.BlockSpec((3 * KE, 32), fixed2),
            pl.BlockSpec((1, 32), fixed2),
            pl.BlockSpec((32, 1), fixed2),
            pl.BlockSpec((1, 1), fixed2),
        ],
        out_specs=pl.BlockSpec((1, 1, 1), lambda b, ids: (b, 0, 0)),
    )
    return pl.pallas_call(
        _tc_body,
        grid_spec=grid_spec,
        out_shape=jax.ShapeDtypeStruct((B, 1, 1), jnp.float32),
    )(doc_ids, qe3, de3, dee3, qee3, idf3, word_adj, ent_adj,
      g1w, g1b, g3w, g3b, g2w, g2b, g4w, g4b,
      w1, b1, w2, b2, w3, b3, w4, b4, w5, b5)


def kernel(qrl_token, doc_token, qrls_ents, docs_ents, doc_ids, word_table,
           ent_table, idf_table, word_adj, ent_adj, G1_W, G1_b, G3_W, G3_b,
           g2_w, g2_b, g4_w, g4_b, W1, b1, W2, b2, W3, b3, W4, b4, W5, b5):
    de_f, qe_f, dee_f, qee_f, idf_f = _sc_gather(
        doc_token.reshape(-1), qrl_token.reshape(-1),
        docs_ents.reshape(-1), qrls_ents.reshape(-1),
        word_table, ent_table, idf_table[:, None])
    out = _tc_call(
        doc_ids,
        qe_f.reshape(B, Lq, DW),
        de_f.reshape(B, Ld, DW),
        dee_f.reshape(B, Ed, DE),
        qee_f.reshape(B, Eq, DE),
        idf_f.reshape(B, Lq, 1),
        word_adj, ent_adj,
        G1_W, G1_b.reshape(6, 1, Lq), G3_W, G3_b.reshape(6, 1, Lq),
        g2_w.reshape(1, 6), g2_b.reshape(1, 6),
        g4_w.reshape(1, 6), g4_b.reshape(1, 6),
        W1, b1.reshape(1, 64), W2, b2.reshape(1, 32), W3, b3.reshape(1, 1),
        W4, b4.reshape(1, 32), W5, b5.reshape(1, 1))
    return out.reshape(B)
